# baseline (speedup 1.0000x reference)
import functools

import jax
import jax.numpy as jnp
import numpy as np
from jax import lax
from jax.experimental import pallas as pl
from jax.experimental.pallas import tpu as pltpu
from jax.experimental.pallas import tpu_sc as plsc

_RCHUNK = 4
_CCHUNK = 4
_NCHUNKS = _RCHUNK * _CCHUNK

_IDXS_NP = np.array(
    [[1, 3, 9, 11, 5, 15, 0, 14, 2, 12, 6, 7, 13, 10, 4, 8],
     [2, 15, 10, 0, 4, 11, 12, 5, 7, 9, 13, 6, 3, 14, 1, 8],
     [5, 7, 10, 0, 1, 4, 2, 13, 12, 6, 3, 8, 11, 14, 15, 9],
     [15, 4, 5, 3, 2, 10, 11, 12, 7, 6, 0, 14, 13, 1, 9, 8],
     [6, 15, 13, 5, 11, 1, 9, 3, 2, 14, 7, 10, 8, 4, 12, 0],
     [8, 3, 1, 9, 13, 7, 12, 15, 2, 4, 0, 10, 11, 5, 6, 14],
     [11, 1, 8, 13, 7, 6, 14, 0, 10, 15, 5, 3, 12, 4, 9, 2],
     [5, 9, 13, 0, 2, 11, 10, 14, 8, 7, 1, 3, 4, 15, 6, 12],
     [11, 2, 12, 8, 3, 10, 13, 5, 4, 15, 0, 9, 14, 7, 6, 1],
     [2, 9, 11, 6, 8, 4, 7, 13, 15, 1, 5, 3, 0, 14, 12, 10],
     [5, 8, 6, 4, 12, 11, 14, 3, 0, 2, 1, 9, 7, 15, 10, 13],
     [15, 8, 9, 2, 11, 7, 14, 12, 0, 6, 1, 3, 13, 10, 4, 5],
     [1, 0, 13, 5, 14, 2, 10, 9, 15, 11, 8, 3, 6, 7, 4, 12],
     [12, 13, 9, 15, 6, 10, 3, 8, 0, 5, 7, 4, 14, 11, 2, 1],
     [0, 12, 5, 10, 15, 11, 9, 2, 1, 7, 4, 3, 6, 14, 8, 13],
     [13, 2, 8, 6, 3, 10, 0, 9, 7, 11, 4, 14, 12, 15, 1, 5]],
    dtype=np.int32,
)

_SRC = [int(_IDXS_NP[k // 16, k % 16]) for k in range(_NCHUNKS * 16)]

_NWORK = 32
_PER_W = (_NCHUNKS * 16) // _NWORK


_SLAB = 12


def _sc_body(protos_hbm, out_hbm, src_smem, buf0, buf1, in_sem, out_sem):
    N, C, H, W = protos_hbm.shape
    Hc, Wc = H // _RCHUNK, W // _CCHUNK
    nslab = C // _SLAB
    bufs = (buf0, buf1)
    for k in range(_NCHUNKS * 16):
        src_smem[k] = np.int32(_SRC[k])
    wid = lax.axis_index("s") * 2 + lax.axis_index("c")
    T = _PER_W * nslab

    def mk(t):
        blk, sl = t // nslab, t % nslab
        k = wid * _PER_W + blk
        m = src_smem[k]
        n = lax.rem(k, 16)
        i = lax.div(k, 16)
        ho = lax.div(i, _CCHUNK) * Hc
        wo = lax.rem(i, _CCHUNK) * Wc
        co = sl * _SLAB
        src = protos_hbm.at[m, pl.ds(co, _SLAB), pl.ds(ho, Hc), pl.ds(wo, Wc)]
        dst = out_hbm.at[n, pl.ds(co, _SLAB), pl.ds(ho, Hc), pl.ds(wo, Wc)]
        return src, dst

    in_d = [None] * T
    out_d = [None] * T
    s0, _ = mk(0)
    in_d[0] = pltpu.make_async_copy(s0, bufs[0], in_sem)
    in_d[0].start()
    for t in range(T):
        if t + 1 < T:
            if t >= 1:
                out_d[t - 1].wait()
            s, _ = mk(t + 1)
            in_d[t + 1] = pltpu.make_async_copy(s, bufs[(t + 1) % 2], in_sem)
            in_d[t + 1].start()
        in_d[t].wait()
        _, dst = mk(t)
        out_d[t] = pltpu.make_async_copy(bufs[t % 2], dst, out_sem)
        out_d[t].start()
    out_d[T - 2].wait()
    out_d[T - 1].wait()


def kernel(protos):
    N, C, H, W = protos.shape
    Hc, Wc = H // _RCHUNK, W // _CCHUNK
    mesh = plsc.VectorSubcoreMesh(core_axis_name="c", subcore_axis_name="s")
    spro = pl.kernel(
        _sc_body,
        out_type=jax.ShapeDtypeStruct((N, C, H, W), protos.dtype),
        mesh=mesh,
        scratch_types=[
            pltpu.SMEM((_NCHUNKS * 16,), jnp.int32),
            pltpu.VMEM((_SLAB, Hc, Wc), jnp.float32),
            pltpu.VMEM((_SLAB, Hc, Wc), jnp.float32),
            pltpu.SemaphoreType.DMA,
            pltpu.SemaphoreType.DMA,
        ],
        compiler_params=pltpu.CompilerParams(use_tc_tiling_on_sc=False),
    )(protos)
    return spro, jnp.asarray(_IDXS_NP)
